# Initial kernel scaffold; baseline (speedup 1.0000x reference)
import jax
import jax.numpy as jnp
from jax.experimental import pallas as pl


def kernel(x, batch, alpha, weight, bias):
    raise NotImplementedError("write your pallas kernel here")


# speedup vs baseline: 7.1177x; 7.1177x over previous
import functools

import jax
import jax.numpy as jnp
from jax.experimental import pallas as pl
from jax.experimental.pallas import tpu as pltpu

N = 100000
D = 512
S = 256
EPS = 1e-09
R = 1000
NB = N // R


def _p1_body(batch_ref, x_ref, alpha_ref, weight_ref, bias_ref, stats_ref,
             acc_ref, cnt_ref):
    i = pl.program_id(0)

    @pl.when(i == 0)
    def _init():
        acc_ref[...] = jnp.zeros_like(acc_ref)
        cnt_ref[...] = jnp.zeros_like(cnt_ref)

    b = batch_ref[...]
    lane = jax.lax.broadcasted_iota(jnp.int32, (R, S), 1)
    oh_bool = b == lane
    oh = oh_bool.astype(jnp.bfloat16)
    xb = x_ref[...].astype(jnp.bfloat16)
    rhs = jnp.concatenate([xb, xb * xb], axis=1)
    acc_ref[...] += jax.lax.dot_general(
        oh, rhs, (((0,), (0,)), ((), ())),
        preferred_element_type=jnp.float32)
    cnt_ref[...] += jnp.sum(oh_bool.astype(jnp.float32), axis=0,
                            keepdims=True)

    @pl.when(i == NB - 1)
    def _finish():
        cnt = cnt_ref[...].reshape(S, 1)
        inv_n = 1.0 / jnp.maximum(cnt, 1.0)
        mu = acc_ref[:, :D] * inv_n
        ex2 = acc_ref[:, D:] * inv_n
        alpha = alpha_ref[...]
        sigma2 = ex2 - (2.0 * alpha - alpha * alpha) * mu * mu + EPS
        a = weight_ref[...] * jax.lax.rsqrt(sigma2)
        bconst = bias_ref[...] - a * alpha * mu
        stats_ref[...] = jnp.concatenate([a, bconst], axis=1)


def _p2_body(batch_ref, x_ref, stats_ref, out_ref):
    b = batch_ref[...]
    lane = jax.lax.broadcasted_iota(jnp.int32, (R, S), 1)
    oh = (b == lane).astype(jnp.bfloat16)
    ab = jax.lax.dot_general(
        oh, stats_ref[...], (((1,), (0,)), ((), ())),
        preferred_element_type=jnp.float32)
    out_ref[...] = ab[:, :D] * x_ref[...] + ab[:, D:]


@jax.jit
def kernel(x, batch, alpha, weight, bias):
    b2 = batch.astype(jnp.int32).reshape(N, 1)
    alpha2 = alpha.reshape(1, D)
    weight2 = weight.reshape(1, D)
    bias2 = bias.reshape(1, D)

    stats = pl.pallas_call(
        _p1_body,
        grid=(NB,),
        in_specs=[
            pl.BlockSpec((R, 1), lambda i: (i, 0)),
            pl.BlockSpec((R, D), lambda i: (i, 0)),
            pl.BlockSpec((1, D), lambda i: (0, 0)),
            pl.BlockSpec((1, D), lambda i: (0, 0)),
            pl.BlockSpec((1, D), lambda i: (0, 0)),
        ],
        out_specs=pl.BlockSpec((S, 2 * D), lambda i: (0, 0)),
        out_shape=jax.ShapeDtypeStruct((S, 2 * D), jnp.float32),
        scratch_shapes=[
            pltpu.VMEM((S, 2 * D), jnp.float32),
            pltpu.VMEM((1, S), jnp.float32),
        ],
    )(b2, x, alpha2, weight2, bias2)

    stats_bf = stats.astype(jnp.bfloat16)

    out = pl.pallas_call(
        _p2_body,
        grid=(NB,),
        in_specs=[
            pl.BlockSpec((R, 1), lambda i: (i, 0)),
            pl.BlockSpec((R, D), lambda i: (i, 0)),
            pl.BlockSpec((S, 2 * D), lambda i: (0, 0)),
        ],
        out_specs=pl.BlockSpec((R, D), lambda i: (i, 0)),
        out_shape=jax.ShapeDtypeStruct((N, D), jnp.float32),
    )(b2, x, stats_bf)
    return out
